# Initial kernel scaffold; baseline (speedup 1.0000x reference)
import jax
import jax.numpy as jnp
from jax.experimental import pallas as pl


def kernel(x, edge_index, W1, att_src1, att_dst1, bias1, W2, att_src2, att_dst2, bias2):
    raise NotImplementedError("write your pallas kernel here")


# speedup vs baseline: 48.0521x; 48.0521x over previous
import functools

import jax
import jax.numpy as jnp
from jax import lax
from jax.experimental import pallas as pl
from jax.experimental.pallas import tpu as pltpu
from jax.experimental.pallas import tpu_sc as plsc

N = 10000
N_PAD = 10240
D = 128
E = 320000
E_TOT = E + N
CHUNK = 128
NC, NS = 2, 16
NW = NC * NS
NCH = -(-E_TOT // (NW * CHUNK))
T_EDGES = NCH * CHUNK
E_PAD = NW * T_EDGES
RPT = N_PAD // NS
F32 = jnp.float32

_mesh = plsc.VectorSubcoreMesh(core_axis_name="c", subcore_axis_name="s")
_sc_params = pltpu.CompilerParams(use_tc_tiling_on_sc=False)


def _edge_sweep_body(src_h, dst_h, feat_h, as_h, ad_h, acc_o, den_o,
                     sh_feat, sh_as, sh_ad, sh_acc, sh_den,
                     v_src, v_dst, v_as, v_ad, v_f, v_w, sem,
                     *, width, tables_in_spmem, heads):
    c = lax.axis_index("c")
    s = lax.axis_index("s")
    wid = c * NS + s
    r0 = s * RPT
    nj = width // 16

    if tables_in_spmem:
        pltpu.sync_copy(feat_h.at[pl.ds(r0, RPT)], sh_feat.at[pl.ds(r0, RPT)])
        pltpu.sync_copy(as_h.at[pl.ds(r0, RPT)], sh_as.at[pl.ds(r0, RPT)])
        pltpu.sync_copy(ad_h.at[pl.ds(r0, RPT)], sh_ad.at[pl.ds(r0, RPT)])

    z16 = jnp.zeros((16,), F32)

    def zrow(k, _):
        for j in range(nj):
            v_f[k, pl.ds(j * 16, 16)] = z16
        v_w[k] = z16
        return 0

    lax.fori_loop(0, CHUNK, zrow, 0)
    done = 0
    while done < RPT:
        n = min(CHUNK, RPT - done)
        pltpu.sync_copy(v_f.at[pl.ds(0, n)], sh_acc.at[pl.ds(r0 + done, n)])
        pltpu.sync_copy(v_w.at[pl.ds(0, n)], sh_den.at[pl.ds(r0 + done, n)])
        done += n
    plsc.subcore_barrier()

    iota16 = lax.iota(jnp.int32, 16)
    ge8 = (iota16 & 8) >> 3
    if heads == 8:
        cols = [2 * j + ge8 for j in range(nj)]
    else:
        cols = [jnp.zeros((16,), jnp.int32)] * nj

    def chunk_body(ci, _):
        base = (wid * NCH + ci) * CHUNK
        pltpu.sync_copy(src_h.at[pl.ds(base, CHUNK)], v_src)
        pltpu.sync_copy(dst_h.at[pl.ds(base, CHUNK)], v_dst)
        if tables_in_spmem:
            pltpu.async_copy(sh_as.at[v_src], v_as, sem).wait()
            pltpu.async_copy(sh_ad.at[v_dst], v_ad, sem).wait()
            pltpu.async_copy(sh_feat.at[v_src], v_f, sem).wait()
        else:
            pltpu.async_copy(as_h.at[v_src], v_as, sem).wait()
            pltpu.async_copy(ad_h.at[v_dst], v_ad, sem).wait()
            pltpu.async_copy(feat_h.at[v_src], v_f, sem).wait()

        def wbody(k, _):
            a = v_as[k] + v_ad[k]
            v_w[k] = jnp.exp(jnp.maximum(a, 0.2 * a))
            return 0

        lax.fori_loop(0, CHUNK, wbody, 0)
        pltpu.sync_copy(v_w, sh_den.at[v_dst], add=True)

        def mbody(k, _):
            w16 = v_w[k]
            for j in range(nj):
                wv = lax.gather(
                    w16, cols[j][:, None],
                    dimension_numbers=lax.GatherDimensionNumbers(
                        offset_dims=(), collapsed_slice_dims=(0,),
                        start_index_map=(0,)),
                    slice_sizes=(1,),
                    mode=lax.GatherScatterMode.PROMISE_IN_BOUNDS)
                v_f[k, pl.ds(j * 16, 16)] = v_f[k, pl.ds(j * 16, 16)] * wv
            return 0

        lax.fori_loop(0, CHUNK, mbody, 0)
        pltpu.sync_copy(v_f, sh_acc.at[v_dst], add=True)
        return 0

    lax.fori_loop(0, NCH, chunk_body, 0)
    plsc.subcore_barrier()

    pltpu.sync_copy(sh_acc.at[pl.ds(r0, RPT)], acc_o.at[c, pl.ds(r0, RPT)])
    pltpu.sync_copy(sh_den.at[pl.ds(r0, RPT)], den_o.at[c, pl.ds(r0, RPT)])


def _make_edge_sweep(width, tables_in_spmem, heads):
    scratch = [
        pltpu.VMEM_SHARED((N_PAD, width), F32) if tables_in_spmem else None,
        pltpu.VMEM_SHARED((N_PAD, 16), F32) if tables_in_spmem else None,
        pltpu.VMEM_SHARED((N_PAD, 16), F32) if tables_in_spmem else None,
        pltpu.VMEM_SHARED((N_PAD, width), F32),
        pltpu.VMEM_SHARED((N_PAD, 16), F32),
        pltpu.VMEM((CHUNK,), jnp.int32),
        pltpu.VMEM((CHUNK,), jnp.int32),
        pltpu.VMEM((CHUNK, 16), F32),
        pltpu.VMEM((CHUNK, 16), F32),
        pltpu.VMEM((CHUNK, width), F32),
        pltpu.VMEM((CHUNK, 16), F32),
        pltpu.SemaphoreType.DMA,
    ]
    live = [i for i, sc in enumerate(scratch) if sc is not None]

    def body(src_h, dst_h, feat_h, as_h, ad_h, acc_o, den_o, *scr):
        full = [None] * len(scratch)
        for i, r in zip(live, scr):
            full[i] = r
        _edge_sweep_body(src_h, dst_h, feat_h, as_h, ad_h, acc_o, den_o,
                         *full,
                         width=width, tables_in_spmem=tables_in_spmem,
                         heads=heads)

    return pl.kernel(
        body,
        out_type=[jax.ShapeDtypeStruct((NC, N_PAD, width), F32),
                  jax.ShapeDtypeStruct((NC, N_PAD, 16), F32)],
        mesh=_mesh,
        compiler_params=_sc_params,
        scratch_types=[scratch[i] for i in live],
    )


def _phase_a(x_pad, W1, att_s, att_d):
    BN = 1024
    grid = (N_PAD // BN,)

    def body(x_ref, w_ref, s_ref, d_ref, h_ref, as_ref, ad_ref):
        h = jnp.dot(x_ref[...], w_ref[...], preferred_element_type=F32)
        h_ref[...] = h
        h3 = h.reshape(BN, 8, 8)
        s8 = jnp.sum(h3 * s_ref[...][None], axis=-1)
        d8 = jnp.sum(h3 * d_ref[...][None], axis=-1)
        z = jnp.zeros_like(s8)
        as_ref[...] = jnp.concatenate([s8, z], axis=1)
        ad_ref[...] = jnp.concatenate([d8, z], axis=1)

    return pl.pallas_call(
        body,
        grid=grid,
        in_specs=[
            pl.BlockSpec((BN, D), lambda i: (i, 0)),
            pl.BlockSpec((D, 64), lambda i: (0, 0)),
            pl.BlockSpec((8, 8), lambda i: (0, 0)),
            pl.BlockSpec((8, 8), lambda i: (0, 0)),
        ],
        out_specs=[
            pl.BlockSpec((BN, 64), lambda i: (i, 0)),
            pl.BlockSpec((BN, 16), lambda i: (i, 0)),
            pl.BlockSpec((BN, 16), lambda i: (i, 0)),
        ],
        out_shape=[
            jax.ShapeDtypeStruct((N_PAD, 64), F32),
            jax.ShapeDtypeStruct((N_PAD, 16), F32),
            jax.ShapeDtypeStruct((N_PAD, 16), F32),
        ],
    )(x_pad, W1, att_s, att_d)


def _phase_c(accp, denp, bias1, W2, att_s2, att_d2):
    BN = 1024
    grid = (N_PAD // BN,)

    def body(a_ref, dn_ref, b_ref, w_ref, s_ref, d_ref, h_ref, as_ref, ad_ref):
        acc = a_ref[0] + a_ref[1]
        den16 = dn_ref[0] + dn_ref[1]
        den8 = den16[:, :8]
        den8 = jnp.where(den8 == 0.0, 1.0, den8)
        denr = jnp.broadcast_to(den8[:, :, None], (BN, 8, 8)).reshape(BN, 64)
        out1 = acc / denr + b_ref[...]
        m = jnp.where(out1 > 0, out1, jnp.exp(jnp.minimum(out1, 0.0)) - 1.0)
        h2 = jnp.dot(m, w_ref[...], preferred_element_type=F32)
        h_ref[...] = h2
        a2s = jnp.sum(h2 * s_ref[...], axis=-1, keepdims=True)
        a2d = jnp.sum(h2 * d_ref[...], axis=-1, keepdims=True)
        z = jnp.zeros((BN, 15), F32)
        as_ref[...] = jnp.concatenate([a2s, z], axis=1)
        ad_ref[...] = jnp.concatenate([a2d, z], axis=1)

    return pl.pallas_call(
        body,
        grid=grid,
        in_specs=[
            pl.BlockSpec((2, BN, 64), lambda i: (0, i, 0)),
            pl.BlockSpec((2, BN, 16), lambda i: (0, i, 0)),
            pl.BlockSpec((1, 64), lambda i: (0, 0)),
            pl.BlockSpec((64, D), lambda i: (0, 0)),
            pl.BlockSpec((1, D), lambda i: (0, 0)),
            pl.BlockSpec((1, D), lambda i: (0, 0)),
        ],
        out_specs=[
            pl.BlockSpec((BN, D), lambda i: (i, 0)),
            pl.BlockSpec((BN, 16), lambda i: (i, 0)),
            pl.BlockSpec((BN, 16), lambda i: (i, 0)),
        ],
        out_shape=[
            jax.ShapeDtypeStruct((N_PAD, D), F32),
            jax.ShapeDtypeStruct((N_PAD, 16), F32),
            jax.ShapeDtypeStruct((N_PAD, 16), F32),
        ],
    )(accp, denp, bias1, W2, att_s2, att_d2)


def _phase_e(accp, denp, bias2):
    BN = 1024
    grid = (N_PAD // BN,)

    def body(a_ref, dn_ref, b_ref, o_ref):
        acc = a_ref[0] + a_ref[1]
        den = dn_ref[0][:, :1] + dn_ref[1][:, :1]
        den = jnp.where(den == 0.0, 1.0, den)
        o_ref[...] = acc / den + b_ref[...]

    return pl.pallas_call(
        body,
        grid=grid,
        in_specs=[
            pl.BlockSpec((2, BN, D), lambda i: (0, i, 0)),
            pl.BlockSpec((2, BN, 16), lambda i: (0, i, 0)),
            pl.BlockSpec((1, D), lambda i: (0, 0)),
        ],
        out_specs=pl.BlockSpec((BN, D), lambda i: (i, 0)),
        out_shape=jax.ShapeDtypeStruct((N_PAD, D), F32),
    )(accp, denp, bias2)


_sweep1 = _make_edge_sweep(64, tables_in_spmem=True, heads=8)
_sweep2 = _make_edge_sweep(128, tables_in_spmem=False, heads=1)


@jax.jit
def kernel(x, edge_index, W1, att_src1, att_dst1, bias1, W2, att_src2,
           att_dst2, bias2):
    x_pad = jnp.pad(x, ((0, N_PAD - N), (0, 0)))
    loop = jnp.arange(N, dtype=edge_index.dtype)
    dummy = jnp.full((E_PAD - E_TOT,), N, dtype=edge_index.dtype)
    src = jnp.concatenate([edge_index[0], loop, dummy])
    dst = jnp.concatenate([edge_index[1], loop, dummy])

    h1, a1s, a1d = _phase_a(x_pad, W1, att_src1.reshape(8, 8),
                            att_dst1.reshape(8, 8))
    acc1, den1 = _sweep1(src, dst, h1, a1s, a1d)
    h2, a2s, a2d = _phase_c(acc1, den1, bias1.reshape(1, 64), W2,
                            att_src2.reshape(1, D), att_dst2.reshape(1, D))
    acc2, den2 = _sweep2(src, dst, h2, a2s, a2d)
    out = _phase_e(acc2, den2, bias2.reshape(1, D))
    return out[:N]


# speedup vs baseline: 56.3241x; 1.1721x over previous
import functools

import jax
import jax.numpy as jnp
from jax import lax
from jax.experimental import pallas as pl
from jax.experimental.pallas import tpu as pltpu
from jax.experimental.pallas import tpu_sc as plsc

N = 10000
N_PAD = 10240
D = 128
E = 320000
E_TOT = E + N
CHUNK = 128
NC, NS = 2, 16
NW = NC * NS
NCH = -(-E_TOT // (NW * CHUNK))
T_EDGES = NCH * CHUNK
E_PAD = NW * T_EDGES
RPT = N_PAD // NS
F32 = jnp.float32

_mesh = plsc.VectorSubcoreMesh(core_axis_name="c", subcore_axis_name="s")
_sc_params = pltpu.CompilerParams(use_tc_tiling_on_sc=False)


def _edge_sweep_body(src_h, dst_h, feat_h, as_h, ad_h, acc_o, den_o,
                     sh_feat, sh_as, sh_ad, sh_acc, sh_den,
                     v_src, v_dst, v_as, v_ad, v_f, v_w, sem,
                     *, width, tables_in_spmem, heads):
    c = lax.axis_index("c")
    s = lax.axis_index("s")
    wid = c * NS + s
    r0 = s * RPT
    nj = width // 16

    if tables_in_spmem:
        pltpu.sync_copy(feat_h.at[pl.ds(r0, RPT)], sh_feat.at[pl.ds(r0, RPT)])
        pltpu.sync_copy(as_h.at[pl.ds(r0, RPT)], sh_as.at[pl.ds(r0, RPT)])
        pltpu.sync_copy(ad_h.at[pl.ds(r0, RPT)], sh_ad.at[pl.ds(r0, RPT)])

    z16 = jnp.zeros((16,), F32)

    @plsc.parallel_loop(0, CHUNK, unroll=4)
    def zrow(k):
        for j in range(nj):
            v_f[k, pl.ds(j * 16, 16)] = z16
        v_w[k] = z16
    done = 0
    while done < RPT:
        n = min(CHUNK, RPT - done)
        pltpu.sync_copy(v_f.at[pl.ds(0, n)], sh_acc.at[pl.ds(r0 + done, n)])
        pltpu.sync_copy(v_w.at[pl.ds(0, n)], sh_den.at[pl.ds(r0 + done, n)])
        done += n
    plsc.subcore_barrier()

    iota16 = lax.iota(jnp.int32, 16)
    ge8 = (iota16 & 8) >> 3
    if heads == 8:
        cols = [2 * j + ge8 for j in range(nj)]
    else:
        cols = [jnp.zeros((16,), jnp.int32)] * nj

    def chunk_body(ci, _):
        base = (wid * NCH + ci) * CHUNK
        pltpu.sync_copy(src_h.at[pl.ds(base, CHUNK)], v_src)
        pltpu.sync_copy(dst_h.at[pl.ds(base, CHUNK)], v_dst)
        if tables_in_spmem:
            pltpu.async_copy(sh_as.at[v_src], v_as, sem).wait()
            pltpu.async_copy(sh_ad.at[v_dst], v_ad, sem).wait()
            pltpu.async_copy(sh_feat.at[v_src], v_f, sem).wait()
        else:
            pltpu.async_copy(as_h.at[v_src], v_as, sem).wait()
            pltpu.async_copy(ad_h.at[v_dst], v_ad, sem).wait()
            pltpu.async_copy(feat_h.at[v_src], v_f, sem).wait()

        @plsc.parallel_loop(0, CHUNK, unroll=4)
        def wbody(k):
            a = v_as[k] + v_ad[k]
            v_w[k] = jnp.exp(jnp.maximum(a, 0.2 * a))
        pltpu.sync_copy(v_w, sh_den.at[v_dst], add=True)

        @plsc.parallel_loop(0, CHUNK, unroll=2)
        def mbody(k):
            w16 = v_w[k]
            for j in range(nj):
                wv = lax.gather(
                    w16, cols[j][:, None],
                    dimension_numbers=lax.GatherDimensionNumbers(
                        offset_dims=(), collapsed_slice_dims=(0,),
                        start_index_map=(0,)),
                    slice_sizes=(1,),
                    mode=lax.GatherScatterMode.PROMISE_IN_BOUNDS)
                v_f[k, pl.ds(j * 16, 16)] = v_f[k, pl.ds(j * 16, 16)] * wv
        pltpu.sync_copy(v_f, sh_acc.at[v_dst], add=True)
        return 0

    lax.fori_loop(0, NCH, chunk_body, 0)
    plsc.subcore_barrier()

    pltpu.sync_copy(sh_acc.at[pl.ds(r0, RPT)], acc_o.at[c, pl.ds(r0, RPT)])
    pltpu.sync_copy(sh_den.at[pl.ds(r0, RPT)], den_o.at[c, pl.ds(r0, RPT)])


def _make_edge_sweep(width, tables_in_spmem, heads):
    scratch = [
        pltpu.VMEM_SHARED((N_PAD, width), F32) if tables_in_spmem else None,
        pltpu.VMEM_SHARED((N_PAD, 16), F32) if tables_in_spmem else None,
        pltpu.VMEM_SHARED((N_PAD, 16), F32) if tables_in_spmem else None,
        pltpu.VMEM_SHARED((N_PAD, width), F32),
        pltpu.VMEM_SHARED((N_PAD, 16), F32),
        pltpu.VMEM((CHUNK,), jnp.int32),
        pltpu.VMEM((CHUNK,), jnp.int32),
        pltpu.VMEM((CHUNK, 16), F32),
        pltpu.VMEM((CHUNK, 16), F32),
        pltpu.VMEM((CHUNK, width), F32),
        pltpu.VMEM((CHUNK, 16), F32),
        pltpu.SemaphoreType.DMA,
    ]
    live = [i for i, sc in enumerate(scratch) if sc is not None]

    def body(src_h, dst_h, feat_h, as_h, ad_h, acc_o, den_o, *scr):
        full = [None] * len(scratch)
        for i, r in zip(live, scr):
            full[i] = r
        _edge_sweep_body(src_h, dst_h, feat_h, as_h, ad_h, acc_o, den_o,
                         *full,
                         width=width, tables_in_spmem=tables_in_spmem,
                         heads=heads)

    return pl.kernel(
        body,
        out_type=[jax.ShapeDtypeStruct((NC, N_PAD, width), F32),
                  jax.ShapeDtypeStruct((NC, N_PAD, 16), F32)],
        mesh=_mesh,
        compiler_params=_sc_params,
        scratch_types=[scratch[i] for i in live],
    )


def _phase_a(x_pad, W1, att_s, att_d):
    BN = 1024
    grid = (N_PAD // BN,)

    def body(x_ref, w_ref, s_ref, d_ref, h_ref, as_ref, ad_ref):
        h = jnp.dot(x_ref[...], w_ref[...], preferred_element_type=F32)
        h_ref[...] = h
        h3 = h.reshape(BN, 8, 8)
        s8 = jnp.sum(h3 * s_ref[...][None], axis=-1)
        d8 = jnp.sum(h3 * d_ref[...][None], axis=-1)
        z = jnp.zeros_like(s8)
        as_ref[...] = jnp.concatenate([s8, z], axis=1)
        ad_ref[...] = jnp.concatenate([d8, z], axis=1)

    return pl.pallas_call(
        body,
        grid=grid,
        in_specs=[
            pl.BlockSpec((BN, D), lambda i: (i, 0)),
            pl.BlockSpec((D, 64), lambda i: (0, 0)),
            pl.BlockSpec((8, 8), lambda i: (0, 0)),
            pl.BlockSpec((8, 8), lambda i: (0, 0)),
        ],
        out_specs=[
            pl.BlockSpec((BN, 64), lambda i: (i, 0)),
            pl.BlockSpec((BN, 16), lambda i: (i, 0)),
            pl.BlockSpec((BN, 16), lambda i: (i, 0)),
        ],
        out_shape=[
            jax.ShapeDtypeStruct((N_PAD, 64), F32),
            jax.ShapeDtypeStruct((N_PAD, 16), F32),
            jax.ShapeDtypeStruct((N_PAD, 16), F32),
        ],
    )(x_pad, W1, att_s, att_d)


def _phase_c(accp, denp, bias1, W2, att_s2, att_d2):
    BN = 1024
    grid = (N_PAD // BN,)

    def body(a_ref, dn_ref, b_ref, w_ref, s_ref, d_ref, h_ref, as_ref, ad_ref):
        acc = a_ref[0] + a_ref[1]
        den16 = dn_ref[0] + dn_ref[1]
        den8 = den16[:, :8]
        den8 = jnp.where(den8 == 0.0, 1.0, den8)
        denr = jnp.broadcast_to(den8[:, :, None], (BN, 8, 8)).reshape(BN, 64)
        out1 = acc / denr + b_ref[...]
        m = jnp.where(out1 > 0, out1, jnp.exp(jnp.minimum(out1, 0.0)) - 1.0)
        h2 = jnp.dot(m, w_ref[...], preferred_element_type=F32)
        h_ref[...] = h2
        a2s = jnp.sum(h2 * s_ref[...], axis=-1, keepdims=True)
        a2d = jnp.sum(h2 * d_ref[...], axis=-1, keepdims=True)
        z = jnp.zeros((BN, 15), F32)
        as_ref[...] = jnp.concatenate([a2s, z], axis=1)
        ad_ref[...] = jnp.concatenate([a2d, z], axis=1)

    return pl.pallas_call(
        body,
        grid=grid,
        in_specs=[
            pl.BlockSpec((2, BN, 64), lambda i: (0, i, 0)),
            pl.BlockSpec((2, BN, 16), lambda i: (0, i, 0)),
            pl.BlockSpec((1, 64), lambda i: (0, 0)),
            pl.BlockSpec((64, D), lambda i: (0, 0)),
            pl.BlockSpec((1, D), lambda i: (0, 0)),
            pl.BlockSpec((1, D), lambda i: (0, 0)),
        ],
        out_specs=[
            pl.BlockSpec((BN, D), lambda i: (i, 0)),
            pl.BlockSpec((BN, 16), lambda i: (i, 0)),
            pl.BlockSpec((BN, 16), lambda i: (i, 0)),
        ],
        out_shape=[
            jax.ShapeDtypeStruct((N_PAD, D), F32),
            jax.ShapeDtypeStruct((N_PAD, 16), F32),
            jax.ShapeDtypeStruct((N_PAD, 16), F32),
        ],
    )(accp, denp, bias1, W2, att_s2, att_d2)


def _phase_e(accp, denp, bias2):
    BN = 1024
    grid = (N_PAD // BN,)

    def body(a_ref, dn_ref, b_ref, o_ref):
        acc = a_ref[0] + a_ref[1]
        den = dn_ref[0][:, :1] + dn_ref[1][:, :1]
        den = jnp.where(den == 0.0, 1.0, den)
        o_ref[...] = acc / den + b_ref[...]

    return pl.pallas_call(
        body,
        grid=grid,
        in_specs=[
            pl.BlockSpec((2, BN, D), lambda i: (0, i, 0)),
            pl.BlockSpec((2, BN, 16), lambda i: (0, i, 0)),
            pl.BlockSpec((1, D), lambda i: (0, 0)),
        ],
        out_specs=pl.BlockSpec((BN, D), lambda i: (i, 0)),
        out_shape=jax.ShapeDtypeStruct((N_PAD, D), F32),
    )(accp, denp, bias2)


_sweep1 = _make_edge_sweep(64, tables_in_spmem=True, heads=8)
_sweep2 = _make_edge_sweep(128, tables_in_spmem=False, heads=1)


@jax.jit
def kernel(x, edge_index, W1, att_src1, att_dst1, bias1, W2, att_src2,
           att_dst2, bias2):
    x_pad = jnp.pad(x, ((0, N_PAD - N), (0, 0)))
    loop = jnp.arange(N, dtype=edge_index.dtype)
    dummy = jnp.full((E_PAD - E_TOT,), N, dtype=edge_index.dtype)
    src = jnp.concatenate([edge_index[0], loop, dummy])
    dst = jnp.concatenate([edge_index[1], loop, dummy])

    h1, a1s, a1d = _phase_a(x_pad, W1, att_src1.reshape(8, 8),
                            att_dst1.reshape(8, 8))
    acc1, den1 = _sweep1(src, dst, h1, a1s, a1d)
    h2, a2s, a2d = _phase_c(acc1, den1, bias1.reshape(1, 64), W2,
                            att_src2.reshape(1, D), att_dst2.reshape(1, D))
    acc2, den2 = _sweep2(src, dst, h2, a2s, a2d)
    out = _phase_e(acc2, den2, bias2.reshape(1, D))
    return out[:N]


# speedup vs baseline: 56.3317x; 1.0001x over previous
import functools

import jax
import jax.numpy as jnp
from jax import lax
from jax.experimental import pallas as pl
from jax.experimental.pallas import tpu as pltpu
from jax.experimental.pallas import tpu_sc as plsc

N = 10000
N_PAD = 10240
D = 128
E = 320000
E_TOT = E + N
CHUNK = 128
NC, NS = 2, 16
NW = NC * NS
NCH = -(-E_TOT // (NW * CHUNK))
T_EDGES = NCH * CHUNK
E_PAD = NW * T_EDGES
RPT = N_PAD // NS
F32 = jnp.float32

_mesh = plsc.VectorSubcoreMesh(core_axis_name="c", subcore_axis_name="s")
_sc_params = pltpu.CompilerParams(use_tc_tiling_on_sc=False)


def _edge_sweep_body(src_h, dst_h, feat_h, as_h, ad_h, acc_o, den_o,
                     sh_feat, sh_as, sh_ad, sh_acc, sh_den,
                     v_src, v_dst, v_as, v_ad, v_f, v_w, sem,
                     *, width, tables_in_spmem, heads):
    c = lax.axis_index("c")
    s = lax.axis_index("s")
    wid = c * NS + s
    r0 = s * RPT
    nj = width // 16

    if tables_in_spmem:
        pltpu.sync_copy(feat_h.at[pl.ds(r0, RPT)], sh_feat.at[pl.ds(r0, RPT)])
        pltpu.sync_copy(as_h.at[pl.ds(r0, RPT)], sh_as.at[pl.ds(r0, RPT)])
        pltpu.sync_copy(ad_h.at[pl.ds(r0, RPT)], sh_ad.at[pl.ds(r0, RPT)])

    z16 = jnp.zeros((16,), F32)

    @plsc.parallel_loop(0, CHUNK, unroll=4)
    def zrow(k):
        for j in range(nj):
            v_f[k, pl.ds(j * 16, 16)] = z16
        v_w[k] = z16
    done = 0
    while done < RPT:
        n = min(CHUNK, RPT - done)
        pltpu.sync_copy(v_f.at[pl.ds(0, n)], sh_acc.at[pl.ds(r0 + done, n)])
        pltpu.sync_copy(v_w.at[pl.ds(0, n)], sh_den.at[pl.ds(r0 + done, n)])
        done += n
    plsc.subcore_barrier()

    iota16 = lax.iota(jnp.int32, 16)
    ge8 = (iota16 & 8) >> 3
    if heads == 8:
        cols = [2 * j + ge8 for j in range(nj)]
    else:
        cols = [jnp.zeros((16,), jnp.int32)] * nj

    def chunk_body(ci, _):
        base = (wid * NCH + ci) * CHUNK
        pltpu.sync_copy(src_h.at[pl.ds(base, CHUNK)], v_src)
        pltpu.sync_copy(dst_h.at[pl.ds(base, CHUNK)], v_dst)
        if tables_in_spmem:
            pltpu.async_copy(sh_as.at[v_src], v_as, sem).wait()
            pltpu.async_copy(sh_ad.at[v_dst], v_ad, sem).wait()
            pltpu.async_copy(sh_feat.at[v_src], v_f, sem).wait()
        else:
            pltpu.async_copy(as_h.at[v_src], v_as, sem).wait()
            pltpu.async_copy(ad_h.at[v_dst], v_ad, sem).wait()
            pltpu.async_copy(feat_h.at[v_src], v_f, sem).wait()

        @plsc.parallel_loop(0, CHUNK, unroll=4)
        def wbody(k):
            a = v_as[k] + v_ad[k]
            v_w[k] = jnp.exp(jnp.maximum(a, 0.2 * a))
        pltpu.sync_copy(v_w, sh_den.at[v_dst], add=True)

        def _bcast(w16, idx):
            return lax.gather(
                w16, idx[:, None],
                dimension_numbers=lax.GatherDimensionNumbers(
                    offset_dims=(), collapsed_slice_dims=(0,),
                    start_index_map=(0,)),
                slice_sizes=(1,),
                mode=lax.GatherScatterMode.PROMISE_IN_BOUNDS)

        @plsc.parallel_loop(0, CHUNK, unroll=4)
        def mbody(k):
            w16 = v_w[k]
            if heads == 1:
                wvs = [_bcast(w16, cols[0])] * nj
            else:
                wvs = [_bcast(w16, cols[j]) for j in range(nj)]
            for j in range(nj):
                v_f[k, pl.ds(j * 16, 16)] = v_f[k, pl.ds(j * 16, 16)] * wvs[j]
        pltpu.sync_copy(v_f, sh_acc.at[v_dst], add=True)
        return 0

    lax.fori_loop(0, NCH, chunk_body, 0)
    plsc.subcore_barrier()

    pltpu.sync_copy(sh_acc.at[pl.ds(r0, RPT)], acc_o.at[c, pl.ds(r0, RPT)])
    pltpu.sync_copy(sh_den.at[pl.ds(r0, RPT)], den_o.at[c, pl.ds(r0, RPT)])


def _make_edge_sweep(width, tables_in_spmem, heads):
    scratch = [
        pltpu.VMEM_SHARED((N_PAD, width), F32) if tables_in_spmem else None,
        pltpu.VMEM_SHARED((N_PAD, 16), F32) if tables_in_spmem else None,
        pltpu.VMEM_SHARED((N_PAD, 16), F32) if tables_in_spmem else None,
        pltpu.VMEM_SHARED((N_PAD, width), F32),
        pltpu.VMEM_SHARED((N_PAD, 16), F32),
        pltpu.VMEM((CHUNK,), jnp.int32),
        pltpu.VMEM((CHUNK,), jnp.int32),
        pltpu.VMEM((CHUNK, 16), F32),
        pltpu.VMEM((CHUNK, 16), F32),
        pltpu.VMEM((CHUNK, width), F32),
        pltpu.VMEM((CHUNK, 16), F32),
        pltpu.SemaphoreType.DMA,
    ]
    live = [i for i, sc in enumerate(scratch) if sc is not None]

    def body(src_h, dst_h, feat_h, as_h, ad_h, acc_o, den_o, *scr):
        full = [None] * len(scratch)
        for i, r in zip(live, scr):
            full[i] = r
        _edge_sweep_body(src_h, dst_h, feat_h, as_h, ad_h, acc_o, den_o,
                         *full,
                         width=width, tables_in_spmem=tables_in_spmem,
                         heads=heads)

    return pl.kernel(
        body,
        out_type=[jax.ShapeDtypeStruct((NC, N_PAD, width), F32),
                  jax.ShapeDtypeStruct((NC, N_PAD, 16), F32)],
        mesh=_mesh,
        compiler_params=_sc_params,
        scratch_types=[scratch[i] for i in live],
    )


def _phase_a(x_pad, W1, att_s, att_d):
    BN = 1024
    grid = (N_PAD // BN,)

    def body(x_ref, w_ref, s_ref, d_ref, h_ref, as_ref, ad_ref):
        h = jnp.dot(x_ref[...], w_ref[...], preferred_element_type=F32)
        h_ref[...] = h
        h3 = h.reshape(BN, 8, 8)
        s8 = jnp.sum(h3 * s_ref[...][None], axis=-1)
        d8 = jnp.sum(h3 * d_ref[...][None], axis=-1)
        z = jnp.zeros_like(s8)
        as_ref[...] = jnp.concatenate([s8, z], axis=1)
        ad_ref[...] = jnp.concatenate([d8, z], axis=1)

    return pl.pallas_call(
        body,
        grid=grid,
        in_specs=[
            pl.BlockSpec((BN, D), lambda i: (i, 0)),
            pl.BlockSpec((D, 64), lambda i: (0, 0)),
            pl.BlockSpec((8, 8), lambda i: (0, 0)),
            pl.BlockSpec((8, 8), lambda i: (0, 0)),
        ],
        out_specs=[
            pl.BlockSpec((BN, 64), lambda i: (i, 0)),
            pl.BlockSpec((BN, 16), lambda i: (i, 0)),
            pl.BlockSpec((BN, 16), lambda i: (i, 0)),
        ],
        out_shape=[
            jax.ShapeDtypeStruct((N_PAD, 64), F32),
            jax.ShapeDtypeStruct((N_PAD, 16), F32),
            jax.ShapeDtypeStruct((N_PAD, 16), F32),
        ],
    )(x_pad, W1, att_s, att_d)


def _phase_c(accp, denp, bias1, W2, att_s2, att_d2):
    BN = 1024
    grid = (N_PAD // BN,)

    def body(a_ref, dn_ref, b_ref, w_ref, s_ref, d_ref, h_ref, as_ref, ad_ref):
        acc = a_ref[0] + a_ref[1]
        den16 = dn_ref[0] + dn_ref[1]
        den8 = den16[:, :8]
        den8 = jnp.where(den8 == 0.0, 1.0, den8)
        denr = jnp.broadcast_to(den8[:, :, None], (BN, 8, 8)).reshape(BN, 64)
        out1 = acc / denr + b_ref[...]
        m = jnp.where(out1 > 0, out1, jnp.exp(jnp.minimum(out1, 0.0)) - 1.0)
        h2 = jnp.dot(m, w_ref[...], preferred_element_type=F32)
        h_ref[...] = h2
        a2s = jnp.sum(h2 * s_ref[...], axis=-1, keepdims=True)
        a2d = jnp.sum(h2 * d_ref[...], axis=-1, keepdims=True)
        z = jnp.zeros((BN, 15), F32)
        as_ref[...] = jnp.concatenate([a2s, z], axis=1)
        ad_ref[...] = jnp.concatenate([a2d, z], axis=1)

    return pl.pallas_call(
        body,
        grid=grid,
        in_specs=[
            pl.BlockSpec((2, BN, 64), lambda i: (0, i, 0)),
            pl.BlockSpec((2, BN, 16), lambda i: (0, i, 0)),
            pl.BlockSpec((1, 64), lambda i: (0, 0)),
            pl.BlockSpec((64, D), lambda i: (0, 0)),
            pl.BlockSpec((1, D), lambda i: (0, 0)),
            pl.BlockSpec((1, D), lambda i: (0, 0)),
        ],
        out_specs=[
            pl.BlockSpec((BN, D), lambda i: (i, 0)),
            pl.BlockSpec((BN, 16), lambda i: (i, 0)),
            pl.BlockSpec((BN, 16), lambda i: (i, 0)),
        ],
        out_shape=[
            jax.ShapeDtypeStruct((N_PAD, D), F32),
            jax.ShapeDtypeStruct((N_PAD, 16), F32),
            jax.ShapeDtypeStruct((N_PAD, 16), F32),
        ],
    )(accp, denp, bias1, W2, att_s2, att_d2)


def _phase_e(accp, denp, bias2):
    BN = 1024
    grid = (N_PAD // BN,)

    def body(a_ref, dn_ref, b_ref, o_ref):
        acc = a_ref[0] + a_ref[1]
        den = dn_ref[0][:, :1] + dn_ref[1][:, :1]
        den = jnp.where(den == 0.0, 1.0, den)
        o_ref[...] = acc / den + b_ref[...]

    return pl.pallas_call(
        body,
        grid=grid,
        in_specs=[
            pl.BlockSpec((2, BN, D), lambda i: (0, i, 0)),
            pl.BlockSpec((2, BN, 16), lambda i: (0, i, 0)),
            pl.BlockSpec((1, D), lambda i: (0, 0)),
        ],
        out_specs=pl.BlockSpec((BN, D), lambda i: (i, 0)),
        out_shape=jax.ShapeDtypeStruct((N_PAD, D), F32),
    )(accp, denp, bias2)


_sweep1 = _make_edge_sweep(64, tables_in_spmem=True, heads=8)
_sweep2 = _make_edge_sweep(128, tables_in_spmem=False, heads=1)


@jax.jit
def kernel(x, edge_index, W1, att_src1, att_dst1, bias1, W2, att_src2,
           att_dst2, bias2):
    x_pad = jnp.pad(x, ((0, N_PAD - N), (0, 0)))
    loop = jnp.arange(N, dtype=edge_index.dtype)
    dummy = jnp.full((E_PAD - E_TOT,), N, dtype=edge_index.dtype)
    src = jnp.concatenate([edge_index[0], loop, dummy])
    dst = jnp.concatenate([edge_index[1], loop, dummy])

    h1, a1s, a1d = _phase_a(x_pad, W1, att_src1.reshape(8, 8),
                            att_dst1.reshape(8, 8))
    acc1, den1 = _sweep1(src, dst, h1, a1s, a1d)
    h2, a2s, a2d = _phase_c(acc1, den1, bias1.reshape(1, 64), W2,
                            att_src2.reshape(1, D), att_dst2.reshape(1, D))
    acc2, den2 = _sweep2(src, dst, h2, a2s, a2d)
    out = _phase_e(acc2, den2, bias2.reshape(1, D))
    return out[:N]


# speedup vs baseline: 65.5319x; 1.1633x over previous
import functools

import jax
import jax.numpy as jnp
from jax import lax
from jax.experimental import pallas as pl
from jax.experimental.pallas import tpu as pltpu
from jax.experimental.pallas import tpu_sc as plsc

N = 10000
N_PAD = 10240
D = 128
E = 320000
E_TOT = E + N
NC, NS = 2, 16
NW = NC * NS
RPT = N_PAD // NS
F32 = jnp.float32


def _even_ceil(a, b):
    n = -(-a // b)
    return n + (n % 2)


CHUNK1 = 96
NCH1 = _even_ceil(E_TOT, NW * CHUNK1)
E_PAD1 = NW * NCH1 * CHUNK1
CHUNK2 = 88
NCH2 = _even_ceil(E_TOT, NW * CHUNK2)
E_PAD2 = NW * NCH2 * CHUNK2

_mesh = plsc.VectorSubcoreMesh(core_axis_name="c", subcore_axis_name="s")
_sc_params = pltpu.CompilerParams(use_tc_tiling_on_sc=False)


def _edge_sweep_body(src_h, dst_h, feat_h, as_h, ad_h, acc_o, den_o, scr,
                     *, chunk, nch, width, tables_in_spmem, heads):
    c = lax.axis_index("c")
    s = lax.axis_index("s")
    wid = c * NS + s
    r0 = s * RPT
    nj = width // 16
    db_a = not tables_in_spmem

    if tables_in_spmem:
        sh_feat, sh_as, sh_ad = scr[0], scr[1], scr[2]
        scr = scr[3:]
    sh_acc, sh_den = scr[0], scr[1]
    v_src = scr[2:4]
    v_dst = scr[4:6]
    v_f = scr[6:8]
    if db_a:
        v_as, v_ad, v_w = scr[8:10], scr[10:12], scr[12]
        sems = scr[13:15]
    else:
        v_as, v_ad, v_w = [scr[8]], [scr[9]], scr[10]
        sems = scr[11:14]

    feat_src = sh_feat if tables_in_spmem else feat_h
    as_src = sh_as if tables_in_spmem else as_h
    ad_src = sh_ad if tables_in_spmem else ad_h

    if tables_in_spmem:
        pltpu.sync_copy(feat_h.at[pl.ds(r0, RPT)], sh_feat.at[pl.ds(r0, RPT)])
        pltpu.sync_copy(as_h.at[pl.ds(r0, RPT)], sh_as.at[pl.ds(r0, RPT)])
        pltpu.sync_copy(ad_h.at[pl.ds(r0, RPT)], sh_ad.at[pl.ds(r0, RPT)])

    z16 = jnp.zeros((16,), F32)

    @plsc.parallel_loop(0, chunk, unroll=4)
    def zrow(k):
        for j in range(nj):
            v_f[0][k, pl.ds(j * 16, 16)] = z16
        v_w[k] = z16

    done = 0
    while done < RPT:
        n = min(chunk, RPT - done)
        pltpu.sync_copy(v_f[0].at[pl.ds(0, n)], sh_acc.at[pl.ds(r0 + done, n)])
        pltpu.sync_copy(v_w.at[pl.ds(0, n)], sh_den.at[pl.ds(r0 + done, n)])
        done += n
    plsc.subcore_barrier()

    iota16 = lax.iota(jnp.int32, 16)
    ge8 = (iota16 & 8) >> 3
    if heads == 8:
        cols = [2 * j + ge8 for j in range(nj)]
    else:
        cols = [jnp.zeros((16,), jnp.int32)] * nj

    def _bcast(w16, idx):
        return lax.gather(
            w16, idx[:, None],
            dimension_numbers=lax.GatherDimensionNumbers(
                offset_dims=(), collapsed_slice_dims=(0,),
                start_index_map=(0,)),
            slice_sizes=(1,),
            mode=lax.GatherScatterMode.PROMISE_IN_BOUNDS)

    def issue(ci, b):
        base = (wid * nch + ci) * chunk
        pltpu.sync_copy(src_h.at[pl.ds(base, chunk)], v_src[b])
        pltpu.sync_copy(dst_h.at[pl.ds(base, chunk)], v_dst[b])
        pltpu.async_copy(feat_src.at[v_src[b]], v_f[b], sems[b])
        if db_a:
            pltpu.async_copy(as_src.at[v_src[b]], v_as[b], sems[b])
            pltpu.async_copy(ad_src.at[v_dst[b]], v_ad[b], sems[b])

    def wait_gathers(b):
        pltpu.make_async_copy(feat_src.at[v_src[b]], v_f[b], sems[b]).wait()
        if db_a:
            pltpu.make_async_copy(as_src.at[v_src[b]], v_as[b], sems[b]).wait()
            pltpu.make_async_copy(ad_src.at[v_dst[b]], v_ad[b], sems[b]).wait()

    issue(0, 0)

    def gbody(g, _):
        for b in (0, 1):
            ci = 2 * g + b
            nb = 1 - b
            ab = b if db_a else 0
            wait_gathers(b)
            if not db_a:
                pltpu.async_copy(as_src.at[v_src[b]], v_as[0], sems[2]).wait()
                pltpu.async_copy(ad_src.at[v_dst[b]], v_ad[0], sems[2]).wait()
            vas, vad, vf = v_as[ab], v_ad[ab], v_f[b]

            @plsc.parallel_loop(0, chunk, unroll=4)
            def wbody(k):
                a = vas[k] + vad[k]
                v_w[k] = jnp.exp(jnp.maximum(a, 0.2 * a))

            pltpu.sync_copy(v_w, sh_den.at[v_dst[b]], add=True)

            @pl.when(ci + 1 < nch)
            def _():
                issue(ci + 1, nb)

            @plsc.parallel_loop(0, chunk, unroll=4)
            def mbody(k):
                w16 = v_w[k]
                wvs = [_bcast(w16, cols[j]) for j in range(nj)]
                for j in range(nj):
                    vf[k, pl.ds(j * 16, 16)] = vf[k, pl.ds(j * 16, 16)] * wvs[j]

            pltpu.sync_copy(vf, sh_acc.at[v_dst[b]], add=True)
        return 0

    lax.fori_loop(0, nch // 2, gbody, 0)
    plsc.subcore_barrier()

    pltpu.sync_copy(sh_acc.at[pl.ds(r0, RPT)], acc_o.at[c, pl.ds(r0, RPT)])
    pltpu.sync_copy(sh_den.at[pl.ds(r0, RPT)], den_o.at[c, pl.ds(r0, RPT)])


def _make_edge_sweep(chunk, nch, width, tables_in_spmem, heads):
    scratch = []
    if tables_in_spmem:
        scratch += [pltpu.VMEM_SHARED((N_PAD, width), F32),
                    pltpu.VMEM_SHARED((N_PAD, 16), F32),
                    pltpu.VMEM_SHARED((N_PAD, 16), F32)]
    scratch += [pltpu.VMEM_SHARED((N_PAD, width), F32),
                pltpu.VMEM_SHARED((N_PAD, 16), F32)]
    scratch += [pltpu.VMEM((chunk,), jnp.int32)] * 4
    scratch += [pltpu.VMEM((chunk, width), F32)] * 2
    if not tables_in_spmem:
        scratch += [pltpu.VMEM((chunk, 16), F32)] * 4
        scratch += [pltpu.VMEM((chunk, 16), F32)]
        scratch += [pltpu.SemaphoreType.DMA] * 2
    else:
        scratch += [pltpu.VMEM((chunk, 16), F32)] * 2
        scratch += [pltpu.VMEM((chunk, 16), F32)]
        scratch += [pltpu.SemaphoreType.DMA] * 3

    def body(src_h, dst_h, feat_h, as_h, ad_h, acc_o, den_o, *scr):
        _edge_sweep_body(src_h, dst_h, feat_h, as_h, ad_h, acc_o, den_o,
                         list(scr), chunk=chunk, nch=nch, width=width,
                         tables_in_spmem=tables_in_spmem, heads=heads)

    return pl.kernel(
        body,
        out_type=[jax.ShapeDtypeStruct((NC, N_PAD, width), F32),
                  jax.ShapeDtypeStruct((NC, N_PAD, 16), F32)],
        mesh=_mesh,
        compiler_params=_sc_params,
        scratch_types=scratch,
    )


def _phase_a(x_pad, W1, att_s, att_d):
    BN = 1024
    grid = (N_PAD // BN,)

    def body(x_ref, w_ref, s_ref, d_ref, h_ref, as_ref, ad_ref):
        h = jnp.dot(x_ref[...], w_ref[...], preferred_element_type=F32)
        h_ref[...] = h
        h3 = h.reshape(BN, 8, 8)
        s8 = jnp.sum(h3 * s_ref[...][None], axis=-1)
        d8 = jnp.sum(h3 * d_ref[...][None], axis=-1)
        z = jnp.zeros_like(s8)
        as_ref[...] = jnp.concatenate([s8, z], axis=1)
        ad_ref[...] = jnp.concatenate([d8, z], axis=1)

    return pl.pallas_call(
        body,
        grid=grid,
        in_specs=[
            pl.BlockSpec((BN, D), lambda i: (i, 0)),
            pl.BlockSpec((D, 64), lambda i: (0, 0)),
            pl.BlockSpec((8, 8), lambda i: (0, 0)),
            pl.BlockSpec((8, 8), lambda i: (0, 0)),
        ],
        out_specs=[
            pl.BlockSpec((BN, 64), lambda i: (i, 0)),
            pl.BlockSpec((BN, 16), lambda i: (i, 0)),
            pl.BlockSpec((BN, 16), lambda i: (i, 0)),
        ],
        out_shape=[
            jax.ShapeDtypeStruct((N_PAD, 64), F32),
            jax.ShapeDtypeStruct((N_PAD, 16), F32),
            jax.ShapeDtypeStruct((N_PAD, 16), F32),
        ],
    )(x_pad, W1, att_s, att_d)


def _phase_c(accp, denp, bias1, W2, att_s2, att_d2):
    BN = 1024
    grid = (N_PAD // BN,)

    def body(a_ref, dn_ref, b_ref, w_ref, s_ref, d_ref, h_ref, as_ref, ad_ref):
        acc = a_ref[0] + a_ref[1]
        den16 = dn_ref[0] + dn_ref[1]
        den8 = den16[:, :8]
        den8 = jnp.where(den8 == 0.0, 1.0, den8)
        denr = jnp.broadcast_to(den8[:, :, None], (BN, 8, 8)).reshape(BN, 64)
        out1 = acc / denr + b_ref[...]
        m = jnp.where(out1 > 0, out1, jnp.exp(jnp.minimum(out1, 0.0)) - 1.0)
        h2 = jnp.dot(m, w_ref[...], preferred_element_type=F32)
        h_ref[...] = h2
        a2s = jnp.sum(h2 * s_ref[...], axis=-1, keepdims=True)
        a2d = jnp.sum(h2 * d_ref[...], axis=-1, keepdims=True)
        z = jnp.zeros((BN, 15), F32)
        as_ref[...] = jnp.concatenate([a2s, z], axis=1)
        ad_ref[...] = jnp.concatenate([a2d, z], axis=1)

    return pl.pallas_call(
        body,
        grid=grid,
        in_specs=[
            pl.BlockSpec((2, BN, 64), lambda i: (0, i, 0)),
            pl.BlockSpec((2, BN, 16), lambda i: (0, i, 0)),
            pl.BlockSpec((1, 64), lambda i: (0, 0)),
            pl.BlockSpec((64, D), lambda i: (0, 0)),
            pl.BlockSpec((1, D), lambda i: (0, 0)),
            pl.BlockSpec((1, D), lambda i: (0, 0)),
        ],
        out_specs=[
            pl.BlockSpec((BN, D), lambda i: (i, 0)),
            pl.BlockSpec((BN, 16), lambda i: (i, 0)),
            pl.BlockSpec((BN, 16), lambda i: (i, 0)),
        ],
        out_shape=[
            jax.ShapeDtypeStruct((N_PAD, D), F32),
            jax.ShapeDtypeStruct((N_PAD, 16), F32),
            jax.ShapeDtypeStruct((N_PAD, 16), F32),
        ],
    )(accp, denp, bias1, W2, att_s2, att_d2)


def _phase_e(accp, denp, bias2):
    BN = 1024
    grid = (N_PAD // BN,)

    def body(a_ref, dn_ref, b_ref, o_ref):
        acc = a_ref[0] + a_ref[1]
        den = dn_ref[0][:, :1] + dn_ref[1][:, :1]
        den = jnp.where(den == 0.0, 1.0, den)
        o_ref[...] = acc / den + b_ref[...]

    return pl.pallas_call(
        body,
        grid=grid,
        in_specs=[
            pl.BlockSpec((2, BN, D), lambda i: (0, i, 0)),
            pl.BlockSpec((2, BN, 16), lambda i: (0, i, 0)),
            pl.BlockSpec((1, D), lambda i: (0, 0)),
        ],
        out_specs=pl.BlockSpec((BN, D), lambda i: (i, 0)),
        out_shape=jax.ShapeDtypeStruct((N_PAD, D), F32),
    )(accp, denp, bias2)


_sweep1 = _make_edge_sweep(CHUNK1, NCH1, 64, tables_in_spmem=True, heads=8)
_sweep2 = _make_edge_sweep(CHUNK2, NCH2, 128, tables_in_spmem=False, heads=1)


@jax.jit
def kernel(x, edge_index, W1, att_src1, att_dst1, bias1, W2, att_src2,
           att_dst2, bias2):
    x_pad = jnp.pad(x, ((0, N_PAD - N), (0, 0)))
    loop = jnp.arange(N, dtype=edge_index.dtype)
    d1 = jnp.full((E_PAD1 - E_TOT,), N, dtype=edge_index.dtype)
    d2 = jnp.full((E_PAD2 - E_TOT,), N, dtype=edge_index.dtype)
    src1 = jnp.concatenate([edge_index[0], loop, d1])
    dst1 = jnp.concatenate([edge_index[1], loop, d1])
    src2 = jnp.concatenate([edge_index[0], loop, d2])
    dst2 = jnp.concatenate([edge_index[1], loop, d2])

    h1, a1s, a1d = _phase_a(x_pad, W1, att_src1.reshape(8, 8),
                            att_dst1.reshape(8, 8))
    acc1, den1 = _sweep1(src1, dst1, h1, a1s, a1d)
    h2, a2s, a2d = _phase_c(acc1, den1, bias1.reshape(1, 64), W2,
                            att_src2.reshape(1, D), att_dst2.reshape(1, D))
    acc2, den2 = _sweep2(src2, dst2, h2, a2s, a2d)
    out = _phase_e(acc2, den2, bias2.reshape(1, D))
    return out[:N]


# speedup vs baseline: 68.4126x; 1.0440x over previous
import functools

import jax
import jax.numpy as jnp
from jax import lax
from jax.experimental import pallas as pl
from jax.experimental.pallas import tpu as pltpu
from jax.experimental.pallas import tpu_sc as plsc

N = 10000
N_PAD = 10240
D = 128
E = 320000
E_TOT = E + N
NC, NS = 2, 16
NW = NC * NS
RPT = N_PAD // NS
F32 = jnp.float32


def _even_ceil(a, b):
    n = -(-a // b)
    return n + (n % 2)


CHUNK1 = 96
NCH1 = _even_ceil(E_TOT, NW * CHUNK1)
E_PAD1 = NW * NCH1 * CHUNK1
CHUNK2 = 88
NCH2 = _even_ceil(E_TOT, NW * CHUNK2)
E_PAD2 = NW * NCH2 * CHUNK2

_mesh = plsc.VectorSubcoreMesh(core_axis_name="c", subcore_axis_name="s")
_sc_params = pltpu.CompilerParams(use_tc_tiling_on_sc=False)


def _edge_sweep_body(src_h, dst_h, feat_h, as_h, ad_h, acc_o, den_o, scr,
                     *, chunk, nch, width, tables_in_spmem, heads):
    c = lax.axis_index("c")
    s = lax.axis_index("s")
    wid = c * NS + s
    r0 = s * RPT
    nj = width // 16
    db_a = not tables_in_spmem

    if tables_in_spmem:
        sh_feat, sh_as, sh_ad = scr[0], scr[1], scr[2]
        scr = scr[3:]
    sh_acc, sh_den = scr[0], scr[1]
    v_src = scr[2:4]
    v_dst = scr[4:6]
    v_f = scr[6:8]
    if db_a:
        v_as, v_ad, v_w = scr[8:10], scr[10:12], scr[12]
        sems = scr[13:15]
    else:
        v_as, v_ad, v_w = [scr[8]], [scr[9]], scr[10]
        sems = scr[11:14]

    feat_src = sh_feat if tables_in_spmem else feat_h
    as_src = sh_as if tables_in_spmem else as_h
    ad_src = sh_ad if tables_in_spmem else ad_h

    if tables_in_spmem:
        pltpu.sync_copy(feat_h.at[pl.ds(r0, RPT)], sh_feat.at[pl.ds(r0, RPT)])
        pltpu.sync_copy(as_h.at[pl.ds(r0, RPT)], sh_as.at[pl.ds(r0, RPT)])
        pltpu.sync_copy(ad_h.at[pl.ds(r0, RPT)], sh_ad.at[pl.ds(r0, RPT)])

    z16 = jnp.zeros((16,), F32)

    @plsc.parallel_loop(0, chunk, unroll=4)
    def zrow(k):
        for j in range(nj):
            v_f[0][k, pl.ds(j * 16, 16)] = z16
        v_w[k] = z16

    done = 0
    while done < RPT:
        n = min(chunk, RPT - done)
        pltpu.sync_copy(v_f[0].at[pl.ds(0, n)], sh_acc.at[pl.ds(r0 + done, n)])
        pltpu.sync_copy(v_w.at[pl.ds(0, n)], sh_den.at[pl.ds(r0 + done, n)])
        done += n
    plsc.subcore_barrier()

    iota16 = lax.iota(jnp.int32, 16)
    ge8 = (iota16 & 8) >> 3
    if heads == 8:
        cols = [2 * j + ge8 for j in range(nj)]
    else:
        cols = [jnp.zeros((16,), jnp.int32)] * nj

    def _bcast(w16, idx):
        return lax.gather(
            w16, idx[:, None],
            dimension_numbers=lax.GatherDimensionNumbers(
                offset_dims=(), collapsed_slice_dims=(0,),
                start_index_map=(0,)),
            slice_sizes=(1,),
            mode=lax.GatherScatterMode.PROMISE_IN_BOUNDS)

    def issue(ci, b):
        base = (wid * nch + ci) * chunk
        pltpu.sync_copy(src_h.at[pl.ds(base, chunk)], v_src[b])
        pltpu.sync_copy(dst_h.at[pl.ds(base, chunk)], v_dst[b])
        pltpu.async_copy(feat_src.at[v_src[b]], v_f[b], sems[b])
        if db_a:
            pltpu.async_copy(as_src.at[v_src[b]], v_as[b], sems[b])
            pltpu.async_copy(ad_src.at[v_dst[b]], v_ad[b], sems[b])

    def wait_gathers(b):
        pltpu.make_async_copy(feat_src.at[v_src[b]], v_f[b], sems[b]).wait()
        if db_a:
            pltpu.make_async_copy(as_src.at[v_src[b]], v_as[b], sems[b]).wait()
            pltpu.make_async_copy(ad_src.at[v_dst[b]], v_ad[b], sems[b]).wait()

    issue(0, 0)

    def gbody(g, _):
        for b in (0, 1):
            ci = 2 * g + b
            nb = 1 - b
            ab = b if db_a else 0
            wait_gathers(b)
            if not db_a:
                pltpu.async_copy(as_src.at[v_src[b]], v_as[0], sems[2]).wait()
                pltpu.async_copy(ad_src.at[v_dst[b]], v_ad[0], sems[2]).wait()
            vas, vad, vf = v_as[ab], v_ad[ab], v_f[b]

            @pl.when(ci + 1 < nch)
            def _():
                issue(ci + 1, nb)

            @plsc.parallel_loop(0, chunk, unroll=4)
            def ebody(k):
                a = vas[k] + vad[k]
                w16 = jnp.exp(jnp.maximum(a, 0.2 * a))
                v_w[k] = w16
                wvs = [_bcast(w16, cols[j]) for j in range(nj)]
                for j in range(nj):
                    vf[k, pl.ds(j * 16, 16)] = vf[k, pl.ds(j * 16, 16)] * wvs[j]

            pltpu.sync_copy(v_w, sh_den.at[v_dst[b]], add=True)
            pltpu.sync_copy(vf, sh_acc.at[v_dst[b]], add=True)
        return 0

    lax.fori_loop(0, nch // 2, gbody, 0)
    plsc.subcore_barrier()

    pltpu.sync_copy(sh_acc.at[pl.ds(r0, RPT)], acc_o.at[c, pl.ds(r0, RPT)])
    pltpu.sync_copy(sh_den.at[pl.ds(r0, RPT)], den_o.at[c, pl.ds(r0, RPT)])


def _make_edge_sweep(chunk, nch, width, tables_in_spmem, heads):
    scratch = []
    if tables_in_spmem:
        scratch += [pltpu.VMEM_SHARED((N_PAD, width), F32),
                    pltpu.VMEM_SHARED((N_PAD, 16), F32),
                    pltpu.VMEM_SHARED((N_PAD, 16), F32)]
    scratch += [pltpu.VMEM_SHARED((N_PAD, width), F32),
                pltpu.VMEM_SHARED((N_PAD, 16), F32)]
    scratch += [pltpu.VMEM((chunk,), jnp.int32)] * 4
    scratch += [pltpu.VMEM((chunk, width), F32)] * 2
    if not tables_in_spmem:
        scratch += [pltpu.VMEM((chunk, 16), F32)] * 4
        scratch += [pltpu.VMEM((chunk, 16), F32)]
        scratch += [pltpu.SemaphoreType.DMA] * 2
    else:
        scratch += [pltpu.VMEM((chunk, 16), F32)] * 2
        scratch += [pltpu.VMEM((chunk, 16), F32)]
        scratch += [pltpu.SemaphoreType.DMA] * 3

    def body(src_h, dst_h, feat_h, as_h, ad_h, acc_o, den_o, *scr):
        _edge_sweep_body(src_h, dst_h, feat_h, as_h, ad_h, acc_o, den_o,
                         list(scr), chunk=chunk, nch=nch, width=width,
                         tables_in_spmem=tables_in_spmem, heads=heads)

    return pl.kernel(
        body,
        out_type=[jax.ShapeDtypeStruct((NC, N_PAD, width), F32),
                  jax.ShapeDtypeStruct((NC, N_PAD, 16), F32)],
        mesh=_mesh,
        compiler_params=_sc_params,
        scratch_types=scratch,
    )


def _phase_a(x_pad, W1, att_s, att_d):
    BN = 1024
    grid = (N_PAD // BN,)

    def body(x_ref, w_ref, s_ref, d_ref, h_ref, as_ref, ad_ref):
        h = jnp.dot(x_ref[...], w_ref[...], preferred_element_type=F32)
        h_ref[...] = h
        h3 = h.reshape(BN, 8, 8)
        s8 = jnp.sum(h3 * s_ref[...][None], axis=-1)
        d8 = jnp.sum(h3 * d_ref[...][None], axis=-1)
        z = jnp.zeros_like(s8)
        as_ref[...] = jnp.concatenate([s8, z], axis=1)
        ad_ref[...] = jnp.concatenate([d8, z], axis=1)

    return pl.pallas_call(
        body,
        grid=grid,
        in_specs=[
            pl.BlockSpec((BN, D), lambda i: (i, 0)),
            pl.BlockSpec((D, 64), lambda i: (0, 0)),
            pl.BlockSpec((8, 8), lambda i: (0, 0)),
            pl.BlockSpec((8, 8), lambda i: (0, 0)),
        ],
        out_specs=[
            pl.BlockSpec((BN, 64), lambda i: (i, 0)),
            pl.BlockSpec((BN, 16), lambda i: (i, 0)),
            pl.BlockSpec((BN, 16), lambda i: (i, 0)),
        ],
        out_shape=[
            jax.ShapeDtypeStruct((N_PAD, 64), F32),
            jax.ShapeDtypeStruct((N_PAD, 16), F32),
            jax.ShapeDtypeStruct((N_PAD, 16), F32),
        ],
    )(x_pad, W1, att_s, att_d)


def _phase_c(accp, denp, bias1, W2, att_s2, att_d2):
    BN = 1024
    grid = (N_PAD // BN,)

    def body(a_ref, dn_ref, b_ref, w_ref, s_ref, d_ref, h_ref, as_ref, ad_ref):
        acc = a_ref[0] + a_ref[1]
        den16 = dn_ref[0] + dn_ref[1]
        den8 = den16[:, :8]
        den8 = jnp.where(den8 == 0.0, 1.0, den8)
        denr = jnp.broadcast_to(den8[:, :, None], (BN, 8, 8)).reshape(BN, 64)
        out1 = acc / denr + b_ref[...]
        m = jnp.where(out1 > 0, out1, jnp.exp(jnp.minimum(out1, 0.0)) - 1.0)
        h2 = jnp.dot(m, w_ref[...], preferred_element_type=F32)
        h_ref[...] = h2
        a2s = jnp.sum(h2 * s_ref[...], axis=-1, keepdims=True)
        a2d = jnp.sum(h2 * d_ref[...], axis=-1, keepdims=True)
        z = jnp.zeros((BN, 15), F32)
        as_ref[...] = jnp.concatenate([a2s, z], axis=1)
        ad_ref[...] = jnp.concatenate([a2d, z], axis=1)

    return pl.pallas_call(
        body,
        grid=grid,
        in_specs=[
            pl.BlockSpec((2, BN, 64), lambda i: (0, i, 0)),
            pl.BlockSpec((2, BN, 16), lambda i: (0, i, 0)),
            pl.BlockSpec((1, 64), lambda i: (0, 0)),
            pl.BlockSpec((64, D), lambda i: (0, 0)),
            pl.BlockSpec((1, D), lambda i: (0, 0)),
            pl.BlockSpec((1, D), lambda i: (0, 0)),
        ],
        out_specs=[
            pl.BlockSpec((BN, D), lambda i: (i, 0)),
            pl.BlockSpec((BN, 16), lambda i: (i, 0)),
            pl.BlockSpec((BN, 16), lambda i: (i, 0)),
        ],
        out_shape=[
            jax.ShapeDtypeStruct((N_PAD, D), F32),
            jax.ShapeDtypeStruct((N_PAD, 16), F32),
            jax.ShapeDtypeStruct((N_PAD, 16), F32),
        ],
    )(accp, denp, bias1, W2, att_s2, att_d2)


def _phase_e(accp, denp, bias2):
    BN = 1024
    grid = (N_PAD // BN,)

    def body(a_ref, dn_ref, b_ref, o_ref):
        acc = a_ref[0] + a_ref[1]
        den = dn_ref[0][:, :1] + dn_ref[1][:, :1]
        den = jnp.where(den == 0.0, 1.0, den)
        o_ref[...] = acc / den + b_ref[...]

    return pl.pallas_call(
        body,
        grid=grid,
        in_specs=[
            pl.BlockSpec((2, BN, D), lambda i: (0, i, 0)),
            pl.BlockSpec((2, BN, 16), lambda i: (0, i, 0)),
            pl.BlockSpec((1, D), lambda i: (0, 0)),
        ],
        out_specs=pl.BlockSpec((BN, D), lambda i: (i, 0)),
        out_shape=jax.ShapeDtypeStruct((N_PAD, D), F32),
    )(accp, denp, bias2)


_sweep1 = _make_edge_sweep(CHUNK1, NCH1, 64, tables_in_spmem=True, heads=8)
_sweep2 = _make_edge_sweep(CHUNK2, NCH2, 128, tables_in_spmem=False, heads=1)


@jax.jit
def kernel(x, edge_index, W1, att_src1, att_dst1, bias1, W2, att_src2,
           att_dst2, bias2):
    x_pad = jnp.pad(x, ((0, N_PAD - N), (0, 0)))
    loop = jnp.arange(N, dtype=edge_index.dtype)
    d1 = jnp.full((E_PAD1 - E_TOT,), N, dtype=edge_index.dtype)
    d2 = jnp.full((E_PAD2 - E_TOT,), N, dtype=edge_index.dtype)
    src1 = jnp.concatenate([edge_index[0], loop, d1])
    dst1 = jnp.concatenate([edge_index[1], loop, d1])
    src2 = jnp.concatenate([edge_index[0], loop, d2])
    dst2 = jnp.concatenate([edge_index[1], loop, d2])

    h1, a1s, a1d = _phase_a(x_pad, W1, att_src1.reshape(8, 8),
                            att_dst1.reshape(8, 8))
    acc1, den1 = _sweep1(src1, dst1, h1, a1s, a1d)
    h2, a2s, a2d = _phase_c(acc1, den1, bias1.reshape(1, 64), W2,
                            att_src2.reshape(1, D), att_dst2.reshape(1, D))
    acc2, den2 = _sweep2(src2, dst2, h2, a2s, a2d)
    out = _phase_e(acc2, den2, bias2.reshape(1, D))
    return out[:N]
